# baseline (speedup 1.0000x reference)
import functools

import jax
import jax.numpy as jnp
from jax import lax
from jax.experimental import pallas as pl
from jax.experimental.pallas import tpu as pltpu
from jax.experimental.pallas import tpu_sc as plsc

_FALSE_NEW = 0.5
_FALSE_LINK = 0.5
_WRONG_LINK = 1.0
_NEG = -1e9

_N = 4096
_B = 512
_G = _N // _B

_NW = 32
_RPW = _N // _NW
_RCHUNK = 2
_WPR = _N // 4


def _top2_body(ana_ref, t_out, s_out, t_acc, s_acc):
    r = pl.program_id(0)
    c = pl.program_id(1)

    @pl.when(c == 0)
    def _init():
        t_acc[...] = jnp.full((_B, _B), _NEG, jnp.float32)
        s_acc[...] = jnp.full((_B, _B), _NEG, jnp.float32)

    @pl.when(c < r)
    def _interior():
        a = ana_ref[...]
        s_acc[...] = jnp.maximum(s_acc[...], jnp.minimum(t_acc[...], a))
        t_acc[...] = jnp.maximum(t_acc[...], a)

    @pl.when(c == r)
    def _diagonal():
        rows = lax.broadcasted_iota(jnp.int32, (_B, _B), 0)
        cols = lax.broadcasted_iota(jnp.int32, (_B, _B), 1)
        a = jnp.where(cols < rows, ana_ref[...], _NEG)
        s_acc[...] = jnp.maximum(s_acc[...], jnp.minimum(t_acc[...], a))
        t_acc[...] = jnp.maximum(t_acc[...], a)

    @pl.when(c == _G - 1)
    def _finalize():
        tl = t_acc[...]
        t_row = jnp.max(tl, axis=1, keepdims=True)
        lane = lax.broadcasted_iota(jnp.int32, (_B, _B), 1)
        first = jnp.min(jnp.where(tl == t_row, lane, _B), axis=1, keepdims=True)
        t2 = jnp.max(jnp.where(lane == first, _NEG, tl), axis=1, keepdims=True)
        s_row = jnp.maximum(jnp.max(s_acc[...], axis=1, keepdims=True), t2)
        t_out[...] = t_row
        s_out[...] = s_row


def _tc_top2(ana):
    def clamp_map(r, c):
        return (r, jnp.minimum(c, r))

    return pl.pallas_call(
        _top2_body,
        grid=(_G, _G),
        in_specs=[pl.BlockSpec((_B, _B), clamp_map)],
        out_specs=[
            pl.BlockSpec((_B, 1), lambda r, c: (r, 0)),
            pl.BlockSpec((_B, 1), lambda r, c: (r, 0)),
        ],
        out_shape=[
            jax.ShapeDtypeStruct((_N, 1), jnp.float32),
            jax.ShapeDtypeStruct((_N, 1), jnp.float32),
        ],
        scratch_shapes=[
            pltpu.VMEM((_B, _B), jnp.float32),
            pltpu.VMEM((_B, _B), jnp.float32),
        ],
        compiler_params=pltpu.CompilerParams(
            dimension_semantics=("arbitrary", "arbitrary"),
        ),
    )(ana)


def _scan_rows(mask_hbm, ana_slv_hbm, sol_out, b_out,
               mbuf0, mbuf1, solv, idxv, cmv, bbuf, bvv, halov, halofv,
               sem0, sem1, gsem):
    wid = lax.axis_index("s") * 2 + lax.axis_index("c")
    base = wid * _RPW
    mbufs = (mbuf0, mbuf1)
    sems = (sem0, sem1)
    lane = lax.iota(jnp.int32, 16)
    pos4 = lane * 4
    halov[pl.ds(0, 16)] = jnp.zeros((16,), jnp.int32)
    halov[pl.ds(32, 16)] = jnp.zeros((16,), jnp.int32)
    halofv[pl.ds(0, 16)] = jnp.zeros((16,), jnp.float32)
    halofv[pl.ds(32, 16)] = jnp.zeros((16,), jnp.float32)

    for b in range(2):
        pltpu.async_copy(
            mask_hbm.at[pl.ds((base + b * _RCHUNK) * _WPR, _RCHUNK * _WPR)],
            mbufs[b], sems[b])

    def row_chunk(k, accvec):
        for b in range(2):
            r0 = base + (2 * k + b) * _RCHUNK
            pltpu.make_async_copy(
                mask_hbm.at[pl.ds(r0 * _WPR, _RCHUNK * _WPR)],
                mbufs[b], sems[b]).wait()
            for rr in range(_RCHUNK):
                wsum = jnp.zeros((16,), jnp.int32)
                psum = jnp.zeros((16,), jnp.int32)
                for ch in range(_WPR // 16):
                    w = mbufs[b][pl.ds(rr * _WPR + ch * 16, 16)]
                    wsum = wsum + w
                    psum = psum + jnp.where(w != 0, pos4 + ch * 64, 0)
                bytev = (jnp.where(wsum >= 256, 1, 0)
                         + jnp.where(wsum >= 65536, 1, 0)
                         + jnp.where(wsum >= 16777216, 1, 0))
                rloc = (2 * k + b) * _RCHUNK + rr
                rmod = rloc % 16
                v = psum + bytev
                for sh in (8, 4, 2, 1):
                    halov[pl.ds(16, 16)] = v
                    v = v + halov[pl.ds(16 - sh, 16)]
                halov[pl.ds(16, 16)] = v
                placed = halov[pl.ds(31 - rmod, 16)]
                accvec = jnp.where(lane == rmod, placed, accvec)

                @pl.when(rmod == 15)
                def _flush(accvec=accvec, rloc=rloc):
                    solv[pl.ds(rloc - 15, 16)] = accvec
            nxt = r0 + 2 * _RCHUNK

            @pl.when(nxt < base + _RPW)
            def _prefetch():
                pltpu.async_copy(
                    mask_hbm.at[pl.ds(nxt * _WPR, _RCHUNK * _WPR)],
                    mbufs[b], sems[b])
        return accvec

    lax.fori_loop(0, _RPW // (2 * _RCHUNK), row_chunk,
                  jnp.zeros((16,), jnp.int32), unroll=False)

    for ch in range(_RPW // 16):
        rows16 = base + ch * 16 + lane
        s16 = solv[pl.ds(ch * 16, 16)]
        idxv[pl.ds(ch * 16, 16)] = rows16 * (_N // 128) + (s16 >> 7)
        cmv[pl.ds(ch * 16, 16)] = s16 & 127
    pltpu.async_copy(ana_slv_hbm.at[idxv], bbuf, gsem).wait()
    for g in range(_RPW // 16):
        cmvec = cmv[pl.ds(g * 16, 16)]
        bacc = jnp.zeros((16,), jnp.float32)
        for l in range(16):
            j = g * 16 + l
            cm = cmvec[l]
            st = (cm >> 4) * 16
            off = cm - st
            w = bbuf[j, pl.ds(st, 16)]
            halofv[pl.ds(16, 16)] = w
            placed = halofv[pl.ds(16 + off - l, 16)]
            bacc = jnp.where(lane == l, placed, bacc)
        bvv[pl.ds(g * 16, 16)] = bacc
    pltpu.sync_copy(solv, sol_out.at[pl.ds(base, _RPW)])
    pltpu.sync_copy(bvv, b_out.at[pl.ds(base, _RPW)])


def _sc_scan(mask_i8, ana_flat):
    mesh = plsc.VectorSubcoreMesh(core_axis_name="c", subcore_axis_name="s")
    f = functools.partial(
        pl.kernel,
        out_type=(
            jax.ShapeDtypeStruct((_N,), jnp.int32),
            jax.ShapeDtypeStruct((_N,), jnp.float32),
        ),
        mesh=mesh,
        scratch_types=[
            pltpu.VMEM((_RCHUNK * _WPR,), jnp.int32),
            pltpu.VMEM((_RCHUNK * _WPR,), jnp.int32),
            pltpu.VMEM((_RPW,), jnp.int32),
            pltpu.VMEM((_RPW,), jnp.int32),
            pltpu.VMEM((_RPW,), jnp.int32),
            pltpu.VMEM((_RPW, 128), jnp.float32),
            pltpu.VMEM((_RPW,), jnp.float32),
            pltpu.VMEM((48,), jnp.int32),
            pltpu.VMEM((48,), jnp.float32),
            pltpu.SemaphoreType.DMA,
            pltpu.SemaphoreType.DMA,
            pltpu.SemaphoreType.DMA,
        ],
    )(_scan_rows)
    return f(mask_i8, ana_flat)


def _combine_rows(t_hbm, s_hbm, sol_hbm, b_hbm, eps_hbm, out_hbm,
                  tv, sv, solv, bv, ev, resv, halof):
    wid = lax.axis_index("s") * 2 + lax.axis_index("c")

    @pl.when(wid == 0)
    def _():
        pltpu.sync_copy(t_hbm, tv)
        pltpu.sync_copy(s_hbm, sv)
        pltpu.sync_copy(sol_hbm, solv)
        pltpu.sync_copy(b_hbm, bv)
        pltpu.sync_copy(eps_hbm, ev)
        lane = lax.iota(jnp.int32, 16)

        def chunk(ch, acc):
            sl = pl.ds(ch * 16, 16)
            t = tv[sl]
            s = sv[sl]
            so = solv[sl]
            b0 = bv[sl]
            e = ev[sl]
            nona = so == (ch * 16 + lane)
            b = jnp.where(nona, e, b0)
            wrong = jnp.where(nona | (t > b) | (s >= t), t, s)
            cost = jnp.where(nona, _FALSE_LINK, _WRONG_LINK)
            c1 = cost * (1.0 + wrong - b)
            c2 = jnp.where(nona, _NEG, _FALSE_NEW * (1.0 + e - b))
            return acc + jnp.maximum(jnp.maximum(c1, c2), 0.0)

        acc = lax.fori_loop(0, _N // 16, chunk, jnp.zeros((16,), jnp.float32),
                            unroll=False)
        halof[pl.ds(0, 16)] = jnp.zeros((16,), jnp.float32)
        halof[pl.ds(32, 16)] = jnp.zeros((16,), jnp.float32)
        v = acc
        for sh in (8, 4, 2, 1):
            halof[pl.ds(16, 16)] = v
            v = v + halof[pl.ds(16 - sh, 16)]
        resv[...] = jnp.where(lane == 0, lax.rev(v, (0,)), 0.0)
        pltpu.sync_copy(resv, out_hbm)


def _sc_combine(t_row, s_row, sol, bval, eps):
    mesh = plsc.VectorSubcoreMesh(core_axis_name="c", subcore_axis_name="s")
    f = functools.partial(
        pl.kernel,
        out_type=jax.ShapeDtypeStruct((16,), jnp.float32),
        mesh=mesh,
        scratch_types=[
            pltpu.VMEM((_N,), jnp.float32),
            pltpu.VMEM((_N,), jnp.float32),
            pltpu.VMEM((_N,), jnp.int32),
            pltpu.VMEM((_N,), jnp.float32),
            pltpu.VMEM((_N,), jnp.float32),
            pltpu.VMEM((16,), jnp.float32),
            pltpu.VMEM((48,), jnp.float32),
        ],
    )(_combine_rows)
    return f(t_row, s_row, sol, bval, eps)


def kernel(eps_scores, ana_scores, solution_mask):
    mask_words = lax.bitcast_convert_type(
        solution_mask.astype(jnp.int8).reshape(_N * _N // 4, 4),
        jnp.int32)
    ana_slivers = ana_scores.reshape(_N * _N // 128, 128)
    t_row, s_row = _tc_top2(ana_scores)
    sol, bval = _sc_scan(mask_words, ana_slivers)
    out = _sc_combine(t_row.reshape(_N), s_row.reshape(_N), sol,
                      bval, eps_scores)
    return out[0]


# speedup vs baseline: 55.7750x; 55.7750x over previous
import jax
import jax.numpy as jnp
from jax import lax
from jax.experimental import pallas as pl
from jax.experimental.pallas import tpu as pltpu

_FALSE_NEW = 0.5
_FALSE_LINK = 0.5
_WRONG_LINK = 1.0
_NEG = -1e9
_NEG2 = -2e9

_N = 4096
_B = 512
_G = _N // _B


def _body(eps_ref, ana_ref, mask_ref, out_ref, wrong_acc, b_acc, nona_acc):
    r = pl.program_id(0)
    c = pl.program_id(1)

    @pl.when((r == 0) & (c == 0))
    def _init_out():
        out_ref[0, 0] = 0.0

    @pl.when(c == 0)
    def _init_acc():
        wrong_acc[...] = jnp.full((_B, _B), _NEG, jnp.float32)
        b_acc[...] = jnp.full((_B, _B), _NEG, jnp.float32)

    @pl.when(c < r)
    def _interior():
        a = ana_ref[...]
        msk = mask_ref[...] != 0
        b_acc[...] = jnp.maximum(b_acc[...], jnp.where(msk, a, _NEG))
        wrong_acc[...] = jnp.maximum(wrong_acc[...], jnp.where(msk, _NEG, a))

    @pl.when(c == r)
    def _diagonal():
        a = ana_ref[...]
        msk = mask_ref[...] != 0
        rows = lax.broadcasted_iota(jnp.int32, (_B, _B), 0)
        cols = lax.broadcasted_iota(jnp.int32, (_B, _B), 1)
        tri = cols < rows
        diag = cols == rows
        eps_col = eps_ref[...]
        scores = jnp.where(diag, eps_col, a)
        b_acc[...] = jnp.maximum(b_acc[...], jnp.where(msk, scores, _NEG))
        wrong_acc[...] = jnp.maximum(wrong_acc[...], jnp.where(tri & ~msk, a, _NEG))
        nona_acc[...] = jnp.where(diag & msk, 1.0, 0.0)

    @pl.when(c == _G - 1)
    def _finalize():
        wrong = jnp.max(wrong_acc[...], axis=1, keepdims=True)
        b = jnp.max(b_acc[...], axis=1, keepdims=True)
        nona = jnp.max(nona_acc[...], axis=1, keepdims=True) > 0.0
        eps_col = eps_ref[...]
        cost = jnp.where(nona, _FALSE_LINK, _WRONG_LINK)
        c1 = cost * (1.0 + wrong - b)
        c2 = jnp.where(nona, _NEG, _FALSE_NEW * (1.0 + eps_col - b))
        loss = jnp.maximum(jnp.maximum(c1, c2), 0.0)
        out_ref[0, 0] += jnp.sum(loss)


def kernel(eps_scores, ana_scores, solution_mask):
    eps2d = eps_scores.reshape(_N, 1)
    mask_i8 = solution_mask.astype(jnp.int8)

    def clamp_map(r, c):
        return (r, jnp.minimum(c, r))

    out = pl.pallas_call(
        _body,
        grid=(_G, _G),
        in_specs=[
            pl.BlockSpec((_B, 1), lambda r, c: (r, 0)),
            pl.BlockSpec((_B, _B), clamp_map),
            pl.BlockSpec((_B, _B), clamp_map),
        ],
        out_specs=pl.BlockSpec((1, 1), lambda r, c: (0, 0), memory_space=pltpu.SMEM),
        out_shape=jax.ShapeDtypeStruct((1, 1), jnp.float32),
        scratch_shapes=[
            pltpu.VMEM((_B, _B), jnp.float32),
            pltpu.VMEM((_B, _B), jnp.float32),
            pltpu.VMEM((_B, _B), jnp.float32),
        ],
        compiler_params=pltpu.CompilerParams(
            dimension_semantics=("arbitrary", "arbitrary"),
        ),
    )(eps2d, ana_scores, mask_i8)
    return out[0, 0]


# speedup vs baseline: 64.8800x; 1.1632x over previous
import jax
import jax.numpy as jnp
from jax import lax
from jax.experimental import pallas as pl
from jax.experimental.pallas import tpu as pltpu

_FALSE_NEW = 0.5
_FALSE_LINK = 0.5
_WRONG_LINK = 1.0
_NEG = -1e9
_NEG2 = -2e9

_N = 4096
_B = 512
_G = _N // _B
_T = _G * (_G + 1) // 2
_OFFS = tuple(r * (r + 1) // 2 for r in range(1, _G))


def _decode(t):
    r = sum(((t >= o).astype(jnp.int32) for o in _OFFS), jnp.int32(0))
    c = t - r * (r + 1) // 2
    return r, c


def _body(eps_ref, ana_ref, mask_ref, out_ref, wrong_acc, b_acc):
    t = pl.program_id(0)
    r, c = _decode(t)

    @pl.when(t == 0)
    def _init_out():
        out_ref[0, 0] = 0.0

    @pl.when(c == 0)
    def _init_acc():
        wrong_acc[...] = jnp.full((_B, _B), _NEG, jnp.float32)
        b_acc[...] = jnp.zeros((_B, _B), jnp.float32)

    @pl.when(c < r)
    def _interior():
        a = ana_ref[...]
        m = mask_ref[...].astype(jnp.float32)
        b_acc[...] += m * a
        wrong_acc[...] = jnp.maximum(wrong_acc[...], a + _NEG2 * m)

    @pl.when(c == r)
    def _diagonal_and_finalize():
        a = ana_ref[...]
        mi = mask_ref[...]
        m = mi.astype(jnp.float32)
        rows = lax.broadcasted_iota(jnp.int32, (_B, _B), 0)
        cols = lax.broadcasted_iota(jnp.int32, (_B, _B), 1)
        tri = cols < rows
        diag = cols == rows
        eps_col = eps_ref[...]
        scores = jnp.where(diag, eps_col, a)
        b_full = b_acc[...] + m * scores
        wrong_full = jnp.maximum(wrong_acc[...],
                                 jnp.where(tri & (mi == 0), a, _NEG))
        nona_col = jnp.max(jnp.where(diag & (mi != 0), 1.0, 0.0),
                           axis=1, keepdims=True)
        wrong = jnp.max(wrong_full, axis=1, keepdims=True)
        b = jnp.sum(b_full, axis=1, keepdims=True)
        nona = nona_col > 0.0
        cost = jnp.where(nona, _FALSE_LINK, _WRONG_LINK)
        c1 = cost * (1.0 + wrong - b)
        c2 = jnp.where(nona, _NEG, _FALSE_NEW * (1.0 + eps_col - b))
        loss = jnp.maximum(jnp.maximum(c1, c2), 0.0)
        out_ref[0, 0] += jnp.sum(loss)


def kernel(eps_scores, ana_scores, solution_mask):
    eps2d = eps_scores.reshape(_N, 1)
    mask_i8 = solution_mask.astype(jnp.int8)

    def tri_map(t):
        r, c = _decode(t)
        return (r, c)

    out = pl.pallas_call(
        _body,
        grid=(_T,),
        in_specs=[
            pl.BlockSpec((_B, 1), lambda t: (_decode(t)[0], 0)),
            pl.BlockSpec((_B, _B), tri_map),
            pl.BlockSpec((_B, _B), tri_map),
        ],
        out_specs=pl.BlockSpec((1, 1), lambda t: (0, 0), memory_space=pltpu.SMEM),
        out_shape=jax.ShapeDtypeStruct((1, 1), jnp.float32),
        scratch_shapes=[
            pltpu.VMEM((_B, _B), jnp.float32),
            pltpu.VMEM((_B, _B), jnp.float32),
        ],
        compiler_params=pltpu.CompilerParams(
            dimension_semantics=("arbitrary",),
        ),
    )(eps2d, ana_scores, mask_i8)
    return out[0, 0]
